# baseline (device time: 50564 ns/iter reference)
import jax
import jax.numpy as jnp
from jax import lax
from jax.experimental import pallas as pl
from jax.experimental.pallas import tpu as pltpu

N_DEV = 4
M_PAD = 576
HALF = M_PAD // 2


def _a2av_body(
    x_ref, d_ref, qsel_ref, out_ref,
    xb_ref, dg_ref, y_ref, ysend_ref,
    d_send_sems, d_recv_sems, send_sems, recv_sems,
):
    me = lax.axis_index("i")
    m, n = x_ref.shape
    dm = d_ref.shape[0]

    barrier_sem = pltpu.get_barrier_semaphore()
    for k in range(1, N_DEV):
        pl.semaphore_signal(
            barrier_sem, inc=1,
            device_id=((me + k) % N_DEV,),
            device_id_type=pl.DeviceIdType.MESH,
        )
    pl.semaphore_wait(barrier_sem, N_DEV - 1)

    dg_ref[pl.ds(me * dm, dm)] = d_ref[...]
    d_sends = []
    for k in range(1, N_DEV):
        sd = pltpu.make_async_remote_copy(
            src_ref=d_ref,
            dst_ref=dg_ref.at[pl.ds(me * dm, dm)],
            send_sem=d_send_sems.at[k - 1],
            recv_sem=d_recv_sems.at[k - 1],
            device_id=((me + k) % N_DEV,),
            device_id_type=pl.DeviceIdType.MESH,
        )
        sd.start()
        d_sends.append(sd)

    xb_ref[...] = x_ref[...].astype(jnp.bfloat16)

    h_iota = lax.broadcasted_iota(jnp.int32, (HALF, m), 0)

    def extract_half(t, h):
        sel = (qsel_ref[pl.ds(t, 1), :] == h_iota + h * HALF).astype(
            jnp.bfloat16
        )
        return jax.lax.dot_general(
            sel, xb_ref[...],
            dimension_numbers=(((1,), (0,)), ((), ())),
            preferred_element_type=jnp.float32,
        ).astype(jnp.bfloat16)

    send_order = [2, 1, 3]
    sends = []
    for k in send_order:
        t = (me + k) % N_DEV
        for h in range(2):
            slot = 2 * (k - 1) + h
            ysend_ref[slot] = extract_half(t, h)
            sx = pltpu.make_async_remote_copy(
                src_ref=ysend_ref.at[slot],
                dst_ref=y_ref.at[2 * me + h],
                send_sem=send_sems.at[slot],
                recv_sem=recv_sems.at[slot],
                device_id=(t,),
                device_id_type=pl.DeviceIdType.MESH,
            )
            sx.start()
            sends.append(sx)

    for h in range(2):
        y_ref[2 * me + h] = extract_half(me, h)

    for k in range(1, N_DEV):
        src = (me - k) % N_DEV
        rd = pltpu.make_async_remote_copy(
            src_ref=d_ref,
            dst_ref=dg_ref.at[pl.ds(src * dm, dm)],
            send_sem=d_send_sems.at[k - 1],
            recv_sem=d_recv_sems.at[k - 1],
            device_id=(src,),
            device_id_type=pl.DeviceIdType.MESH,
        )
        rd.wait_recv()
    cnt = [
        jnp.sum((dg_ref[pl.ds(s * dm, dm)] == me).astype(jnp.int32))
        for s in range(N_DEV)
    ]
    off = [jnp.int32(0)]
    for s in range(1, N_DEV):
        off.append(off[s - 1] + cnt[s - 1])

    def scalar_pick(s, vals):
        r = vals[0]
        for i in range(1, N_DEV):
            r = jnp.where(s == i, vals[i], r)
        return r

    j_iota = lax.broadcasted_iota(jnp.int32, (m, HALF), 0)
    q_iota = lax.broadcasted_iota(jnp.int32, (m, HALF), 1)

    def place(src):
        c_s = scalar_pick(src, cnt)
        o_s = scalar_pick(src, off)
        acc = None
        for h in range(2):
            g = h * HALF + q_iota
            sel = ((j_iota == o_s + g) & (g < c_s)).astype(jnp.bfloat16)
            part = jax.lax.dot_general(
                sel, y_ref[2 * src + h],
                dimension_numbers=(((1,), (0,)), ((), ())),
                preferred_element_type=jnp.float32,
            )
            acc = part if acc is None else acc + part
        return acc.astype(jnp.bfloat16)

    out_ref[...] = place(me)

    for k in send_order:
        src = (me - k) % N_DEV
        for h in range(2):
            slot = 2 * (k - 1) + h
            rx = pltpu.make_async_remote_copy(
                src_ref=ysend_ref.at[slot],
                dst_ref=y_ref.at[2 * src + h],
                send_sem=send_sems.at[slot],
                recv_sem=recv_sems.at[slot],
                device_id=(src,),
                device_id_type=pl.DeviceIdType.MESH,
            )
            rx.wait_recv()
        out_ref[...] += place(src)

    for sd in d_sends:
        sd.wait_send()
    for sx in sends:
        sx.wait_send()


def kernel(x, dest):
    m, n = x.shape
    dm, dn = 16, 128

    eq = dest[None, :] == jnp.arange(N_DEV, dtype=dest.dtype)[:, None]
    ranks = jnp.cumsum(eq.astype(jnp.int32), axis=1) - 1
    qsel = jnp.where(eq, ranks, -1).astype(jnp.int32)

    return pl.pallas_call(
        _a2av_body,
        out_shape=jax.ShapeDtypeStruct((m, n), jnp.bfloat16),
        in_specs=[
            pl.BlockSpec(memory_space=pltpu.VMEM),
            pl.BlockSpec(memory_space=pltpu.VMEM),
            pl.BlockSpec(memory_space=pltpu.VMEM),
        ],
        out_specs=pl.BlockSpec(memory_space=pltpu.VMEM),
        scratch_shapes=[
            pltpu.VMEM((m, n), jnp.bfloat16),
            pltpu.VMEM((N_DEV * dm, dn), jnp.int32),
            pltpu.VMEM((2 * N_DEV, HALF, n), jnp.bfloat16),
            pltpu.VMEM((2 * (N_DEV - 1), HALF, n), jnp.bfloat16),
            pltpu.SemaphoreType.DMA((N_DEV - 1,)),
            pltpu.SemaphoreType.DMA((N_DEV - 1,)),
            pltpu.SemaphoreType.DMA((2 * (N_DEV - 1),)),
            pltpu.SemaphoreType.DMA((2 * (N_DEV - 1),)),
        ],
        compiler_params=pltpu.CompilerParams(collective_id=0),
    )(x, dest.reshape(dm, dn), qsel)


# device time: 48572 ns/iter; 1.0410x vs baseline; 1.0410x over previous
import jax
import jax.numpy as jnp
from jax import lax
from jax.experimental import pallas as pl
from jax.experimental.pallas import tpu as pltpu

N_DEV = 4
M_PAD = 576


def _a2av_body(
    x_ref, d_ref, qsel_ref, out_ref,
    xb_ref, dg_ref, y_ref, ysend_ref,
    d_send_sems, d_recv_sems, send_sems, recv_sems,
):
    me = lax.axis_index("i")
    m, n = x_ref.shape
    dm = d_ref.shape[0]

    barrier_sem = pltpu.get_barrier_semaphore()
    for k in range(1, N_DEV):
        pl.semaphore_signal(
            barrier_sem, inc=1,
            device_id=((me + k) % N_DEV,),
            device_id_type=pl.DeviceIdType.MESH,
        )
    pl.semaphore_wait(barrier_sem, N_DEV - 1)

    dg_ref[pl.ds(me * dm, dm)] = d_ref[...]
    d_sends = []
    for k in range(1, N_DEV):
        sd = pltpu.make_async_remote_copy(
            src_ref=d_ref,
            dst_ref=dg_ref.at[pl.ds(me * dm, dm)],
            send_sem=d_send_sems.at[k - 1],
            recv_sem=d_recv_sems.at[k - 1],
            device_id=((me + k) % N_DEV,),
            device_id_type=pl.DeviceIdType.MESH,
        )
        sd.start()
        d_sends.append(sd)

    xb_ref[...] = x_ref[...].astype(jnp.bfloat16)

    q_iota = lax.broadcasted_iota(jnp.int32, (M_PAD, m), 0)

    def extract(t):
        sel = (qsel_ref[pl.ds(t, 1), :] == q_iota).astype(jnp.bfloat16)
        return jax.lax.dot_general(
            sel, xb_ref[...],
            dimension_numbers=(((1,), (0,)), ((), ())),
            preferred_element_type=jnp.float32,
        ).astype(jnp.bfloat16)

    sends = []
    for k in range(1, N_DEV):
        t = (me + k) % N_DEV
        ysend_ref[k - 1] = extract(t)
        sx = pltpu.make_async_remote_copy(
            src_ref=ysend_ref.at[k - 1],
            dst_ref=y_ref.at[me],
            send_sem=send_sems.at[k - 1],
            recv_sem=recv_sems.at[k - 1],
            device_id=(t,),
            device_id_type=pl.DeviceIdType.MESH,
        )
        sx.start()
        sends.append(sx)

    y_ref[me] = extract(me)

    for k in range(1, N_DEV):
        src = (me - k) % N_DEV
        rd = pltpu.make_async_remote_copy(
            src_ref=d_ref,
            dst_ref=dg_ref.at[pl.ds(src * dm, dm)],
            send_sem=d_send_sems.at[k - 1],
            recv_sem=d_recv_sems.at[k - 1],
            device_id=(src,),
            device_id_type=pl.DeviceIdType.MESH,
        )
        rd.wait_recv()
    cnt = [
        jnp.sum((dg_ref[pl.ds(s * dm, dm)] == me).astype(jnp.int32))
        for s in range(N_DEV)
    ]
    off = [jnp.int32(0)]
    for s in range(1, N_DEV):
        off.append(off[s - 1] + cnt[s - 1])

    def scalar_pick(s, vals):
        r = vals[0]
        for i in range(1, N_DEV):
            r = jnp.where(s == i, vals[i], r)
        return r

    j_iota = lax.broadcasted_iota(jnp.int32, (m, M_PAD), 0)
    q_iota2 = lax.broadcasted_iota(jnp.int32, (m, M_PAD), 1)

    def place(src):
        c_s = scalar_pick(src, cnt)
        o_s = scalar_pick(src, off)
        sel = ((j_iota == o_s + q_iota2) & (q_iota2 < c_s)).astype(
            jnp.bfloat16
        )
        return jax.lax.dot_general(
            sel, y_ref[src],
            dimension_numbers=(((1,), (0,)), ((), ())),
            preferred_element_type=jnp.float32,
        ).astype(jnp.bfloat16)

    out_ref[...] = place(me)

    for k in range(1, N_DEV):
        src = (me - k) % N_DEV
        rx = pltpu.make_async_remote_copy(
            src_ref=ysend_ref.at[k - 1],
            dst_ref=y_ref.at[src],
            send_sem=send_sems.at[k - 1],
            recv_sem=recv_sems.at[k - 1],
            device_id=(src,),
            device_id_type=pl.DeviceIdType.MESH,
        )
        rx.wait_recv()
        out_ref[...] += place(src)

    for sd in d_sends:
        sd.wait_send()
    for sx in sends:
        sx.wait_send()


def kernel(x, dest):
    m, n = x.shape
    dm, dn = 16, 128

    eq = dest[None, :] == jnp.arange(N_DEV, dtype=dest.dtype)[:, None]
    ranks = jnp.cumsum(eq.astype(jnp.int32), axis=1) - 1
    qsel = jnp.where(eq, ranks, -1).astype(jnp.int32)

    return pl.pallas_call(
        _a2av_body,
        out_shape=jax.ShapeDtypeStruct((m, n), jnp.bfloat16),
        in_specs=[
            pl.BlockSpec(memory_space=pltpu.VMEM),
            pl.BlockSpec(memory_space=pltpu.VMEM),
            pl.BlockSpec(memory_space=pltpu.VMEM),
        ],
        out_specs=pl.BlockSpec(memory_space=pltpu.VMEM),
        scratch_shapes=[
            pltpu.VMEM((m, n), jnp.bfloat16),
            pltpu.VMEM((N_DEV * dm, dn), jnp.int32),
            pltpu.VMEM((N_DEV, M_PAD, n), jnp.bfloat16),
            pltpu.VMEM((N_DEV - 1, M_PAD, n), jnp.bfloat16),
            pltpu.SemaphoreType.DMA((N_DEV - 1,)),
            pltpu.SemaphoreType.DMA((N_DEV - 1,)),
            pltpu.SemaphoreType.DMA((N_DEV - 1,)),
            pltpu.SemaphoreType.DMA((N_DEV - 1,)),
        ],
        compiler_params=pltpu.CompilerParams(collective_id=0),
    )(x, dest.reshape(dm, dn), qsel)
